# baseline (speedup 1.0000x reference)
import functools

import jax
import jax.numpy as jnp
from jax import lax
from jax.experimental import pallas as pl
from jax.experimental.pallas import tpu as pltpu
from jax.experimental.pallas import tpu_sc as plsc

N = 10000
NP = 10240
EP = 163840
NTILES = 32
EDGES_PER_TILE = EP // NTILES

_MESH = plsc.VectorSubcoreMesh(core_axis_name="c", subcore_axis_name="s")
_SC_PARAMS = pltpu.CompilerParams(needs_layout_passes=False,
                                  use_tc_tiling_on_sc=False)


@functools.partial(
    pl.kernel,
    mesh=_MESH,
    out_type=jax.ShapeDtypeStruct((NTILES * NP,), jnp.float32),
    compiler_params=_SC_PARAMS,
    scratch_types=[
        pltpu.VMEM((EDGES_PER_TILE,), jnp.int32),
        pltpu.VMEM((EDGES_PER_TILE,), jnp.float32),
        pltpu.VMEM((NP,), jnp.float32),
    ],
)
def _deg_partials(dst_hbm, ew_hbm, part_hbm, dstv, ewv, degv):
    wid = lax.axis_index("c") * 16 + lax.axis_index("s")
    base = wid * EDGES_PER_TILE
    pltpu.sync_copy(dst_hbm.at[pl.ds(base, EDGES_PER_TILE)], dstv)
    pltpu.sync_copy(ew_hbm.at[pl.ds(base, EDGES_PER_TILE)], ewv)

    def _zero(i, _):
        degv[pl.ds(i * 16, 16)] = jnp.zeros((16,), jnp.float32)
        return 0

    lax.fori_loop(0, NP // 16, _zero, 0)

    def _acc(k, _):
        idx = dstv[pl.ds(k * 16, 16)]
        w = ewv[pl.ds(k * 16, 16)]
        plsc.addupdate_scatter(degv, [idx], w)
        return 0

    lax.fori_loop(0, EDGES_PER_TILE // 16, _acc, 0)
    pltpu.sync_copy(degv, part_hbm.at[pl.ds(wid * NP, NP)])


def _dis_body(p_ref, dis_ref, d2_ref):
    deg = 1.0 + jnp.sum(p_ref[...], axis=0, keepdims=True)
    d2_ref[...] = 1.0 / deg
    dis_ref[...] = lax.rsqrt(deg)


def _dis_from_partials(partials_padded):
    return pl.pallas_call(
        _dis_body,
        out_shape=(
            jax.ShapeDtypeStruct((1, NP), jnp.float32),
            jax.ShapeDtypeStruct((1, NP), jnp.float32),
        ),
    )(partials_padded)


@functools.partial(
    pl.kernel,
    mesh=_MESH,
    out_type=jax.ShapeDtypeStruct((EP,), jnp.float32),
    compiler_params=_SC_PARAMS,
    scratch_types=[
        pltpu.VMEM((NP,), jnp.float32),
        pltpu.VMEM((EDGES_PER_TILE,), jnp.int32),
        pltpu.VMEM((EDGES_PER_TILE,), jnp.int32),
        pltpu.VMEM((EDGES_PER_TILE,), jnp.float32),
        pltpu.VMEM((EDGES_PER_TILE,), jnp.float32),
    ],
)
def _edge_norm(src_hbm, dst_hbm, ew_hbm, dis_hbm, norm_hbm,
               disv, srcv, dstv, ewv, normv):
    wid = lax.axis_index("c") * 16 + lax.axis_index("s")
    base = wid * EDGES_PER_TILE
    pltpu.sync_copy(dis_hbm, disv)
    pltpu.sync_copy(src_hbm.at[pl.ds(base, EDGES_PER_TILE)], srcv)
    pltpu.sync_copy(dst_hbm.at[pl.ds(base, EDGES_PER_TILE)], dstv)
    pltpu.sync_copy(ew_hbm.at[pl.ds(base, EDGES_PER_TILE)], ewv)

    def _body(k, _):
        s = srcv[pl.ds(k * 16, 16)]
        d = dstv[pl.ds(k * 16, 16)]
        w = ewv[pl.ds(k * 16, 16)]
        a = plsc.load_gather(disv, [s])
        b = plsc.load_gather(disv, [d])
        normv[pl.ds(k * 16, 16)] = a * w * b
        return 0

    lax.fori_loop(0, EDGES_PER_TILE // 16, _body, 0)
    pltpu.sync_copy(normv, norm_hbm.at[pl.ds(base, EDGES_PER_TILE)])


_NBLK = 80
_BLK = 128
_ROWS_PER_TILE = NP // 16


def _make_agg128():
    F = 128

    @functools.partial(
        pl.kernel,
        mesh=_MESH,
        out_type=(
            jax.ShapeDtypeStruct((NP, F), jnp.float32),
            jax.ShapeDtypeStruct((NP, F), jnp.float32),
        ),
        compiler_params=_SC_PARAMS,
        scratch_types=[
            pltpu.VMEM((4, 3, _BLK), jnp.int32),
            pltpu.VMEM((_BLK, F), jnp.float32),
            pltpu.VMEM((_BLK, F), jnp.float32),
            pltpu.VMEM_SHARED((NP, F), jnp.float32),
            pltpu.SemaphoreType.DMA,
            pltpu.SemaphoreType.DMA,
            pltpu.SemaphoreType.DMA,
            pltpu.SemaphoreType.DMA,
            pltpu.SemaphoreType.DMA,
            pltpu.SemaphoreType.DMA,
            pltpu.SemaphoreType.DMA,
            pltpu.SemaphoreType.DMA,
        ],
    )
    def _agg(hL, hR, edata, zeros_hbm, outL, outR,
             eb, rows0, rows1, acc, e0, e1, e2, e3, g0, g1, s0, s1):
        cid = lax.axis_index("c")
        sid = lax.axis_index("s")
        rsl = pl.ds(sid * _ROWS_PER_TILE, _ROWS_PER_TILE)
        pltpu.sync_copy(zeros_hbm.at[rsl], acc.at[rsl])
        plsc.subcore_barrier()

        rows = (rows0, rows1)
        esem = (e0, e1, e2, e3)
        gsem = (g0, g1)
        ssem = (s0, s1)
        base = sid * _NBLK

        def _process(h_hbm):
            def estart(j, s):
                pltpu.async_copy(edata.at[base + j], eb.at[s], esem[s])

            def ewait(s):
                pltpu.make_async_copy(
                    edata.at[base], eb.at[s], esem[s]).wait()

            def gstart(r, s):
                pltpu.async_copy(h_hbm.at[eb.at[s, 0]], rows[r], gsem[r])

            def gwait(r, s):
                pltpu.make_async_copy(
                    h_hbm.at[eb.at[s, 0]], rows[r], gsem[r]).wait()

            def sstart(r, s):
                pltpu.async_copy(rows[r], acc.at[eb.at[s, 1]], ssem[r],
                                 add=True)

            def swait(r, s):
                pltpu.make_async_copy(
                    rows[r], acc.at[eb.at[s, 1]], ssem[r]).wait()

            def scale(r, s):
                rp = rows[r]
                iota = lax.iota(jnp.int32, 16)

                def _scale(g, _):
                    eidx = iota + g * 16
                    w = plsc.bitcast(eb[s, 2, pl.ds(g * 16, 16)], jnp.float32)
                    fv = jnp.zeros((16,), jnp.int32)
                    for f in range(F):
                        v = plsc.load_gather(rp, [eidx, fv])
                        plsc.store_scatter(rp, [eidx, fv], v * w)
                        if f + 1 < F:
                            fv = fv + 1
                    return 0

                lax.fori_loop(0, _BLK // 16, _scale, 0)

            for s in range(3):
                estart(s, s)
            ewait(0)
            gstart(0, 0)

            def _outer(i, _):
                for jp in range(4):
                    r = jp % 2
                    q = 1 - r
                    j = 4 * i + jp
                    if jp == 0:
                        @pl.when(i >= 1)
                        def _wq():
                            swait(q, (jp + 3) % 4)
                    else:
                        swait(q, (jp + 3) % 4)
                    if jp == 0:
                        estart(j + 3, 3)
                    else:
                        @pl.when(j + 3 < _NBLK)
                        def _st():
                            estart(j + 3, (jp + 3) % 4)
                    if jp < 3:
                        ewait(jp + 1)
                        gstart(q, jp + 1)
                    else:
                        @pl.when(i < _NBLK // 4 - 1)
                        def _g0():
                            ewait(0)
                            gstart(q, 0)
                    gwait(r, jp)
                    scale(r, jp)
                    sstart(r, jp)
                return 0

            lax.fori_loop(0, _NBLK // 4, _outer, 0)
            swait((_NBLK - 1) % 2, (_NBLK - 1) % 4)

        @pl.when(cid == 0)
        def _left():
            _process(hL)

        @pl.when(cid == 1)
        def _right():
            _process(hR)

        plsc.subcore_barrier()

        @pl.when(cid == 0)
        def _outl():
            pltpu.sync_copy(acc.at[rsl], outL.at[rsl])

        @pl.when(cid == 1)
        def _outr():
            pltpu.sync_copy(acc.at[rsl], outR.at[rsl])

    return _agg


def _make_agg32():
    F = 32
    ROWS3 = _NBLK * 3

    @functools.partial(
        pl.kernel,
        mesh=_MESH,
        out_type=(
            jax.ShapeDtypeStruct((NP, F), jnp.float32),
            jax.ShapeDtypeStruct((NP, F), jnp.float32),
        ),
        compiler_params=_SC_PARAMS,
        scratch_types=[
            pltpu.VMEM((ROWS3, _BLK), jnp.int32),
            pltpu.VMEM((_BLK, F), jnp.float32),
            pltpu.VMEM((_BLK, F), jnp.float32),
            pltpu.VMEM_SHARED((NP, F), jnp.float32),
            pltpu.SemaphoreType.DMA,
            pltpu.SemaphoreType.DMA,
            pltpu.SemaphoreType.DMA,
            pltpu.SemaphoreType.DMA,
        ],
    )
    def _agg(hL, hR, edata2, zeros_hbm, outL, outR,
             eball, rows0, rows1, acc, g0, g1, s0, s1):
        cid = lax.axis_index("c")
        sid = lax.axis_index("s")
        rsl = pl.ds(sid * _ROWS_PER_TILE, _ROWS_PER_TILE)
        pltpu.sync_copy(zeros_hbm.at[rsl], acc.at[rsl])
        pltpu.sync_copy(edata2.at[pl.ds(sid * ROWS3, ROWS3)], eball)
        plsc.subcore_barrier()

        rows = (rows0, rows1)
        gsem = (g0, g1)
        ssem = (s0, s1)

        def _process(h_hbm):
            def gstart(j, r):
                pltpu.async_copy(h_hbm.at[eball.at[3 * j]], rows[r], gsem[r])

            def gwait(r):
                pltpu.make_async_copy(
                    h_hbm.at[eball.at[0]], rows[r], gsem[r]).wait()

            def sstart(j, r):
                pltpu.async_copy(rows[r], acc.at[eball.at[3 * j + 1]],
                                 ssem[r], add=True)

            def swait(r):
                pltpu.make_async_copy(
                    rows[r], acc.at[eball.at[1]], ssem[r]).wait()

            def scale(j, r):
                rp = rows[r]
                iota = lax.iota(jnp.int32, 16)

                def _scale(g, _):
                    eidx = iota + g * 16
                    w = plsc.bitcast(eball[3 * j + 2, pl.ds(g * 16, 16)],
                                     jnp.float32)
                    fv = jnp.zeros((16,), jnp.int32)
                    for f in range(F):
                        v = plsc.load_gather(rp, [eidx, fv])
                        plsc.store_scatter(rp, [eidx, fv], v * w)
                        if f + 1 < F:
                            fv = fv + 1
                    return 0

                lax.fori_loop(0, _BLK // 16, _scale, 0)

            gstart(0, 0)

            def _outer(i, _):
                @pl.when(i >= 1)
                def _w1():
                    swait(1)
                gstart(2 * i + 1, 1)
                gwait(0)
                scale(2 * i, 0)
                sstart(2 * i, 0)
                swait(0)
                @pl.when(i < _NBLK // 2 - 1)
                def _g0():
                    gstart(2 * i + 2, 0)
                gwait(1)
                scale(2 * i + 1, 1)
                sstart(2 * i + 1, 1)
                return 0

            lax.fori_loop(0, _NBLK // 2, _outer, 0)
            swait(1)

        @pl.when(cid == 0)
        def _left():
            _process(hL)

        @pl.when(cid == 1)
        def _right():
            _process(hR)

        plsc.subcore_barrier()

        @pl.when(cid == 0)
        def _outl():
            pltpu.sync_copy(acc.at[rsl], outL.at[rsl])

        @pl.when(cid == 1)
        def _outr():
            pltpu.sync_copy(acc.at[rsl], outR.at[rsl])

    return _agg


_AGG128 = _make_agg128()
_AGG32 = _make_agg32()


def _mm_body(a_ref, b_ref, o_ref):
    @pl.when(pl.program_id(1) == 0)
    def _init():
        o_ref[...] = jnp.zeros_like(o_ref)

    o_ref[...] += jnp.dot(a_ref[...], b_ref[...],
                          preferred_element_type=jnp.float32)


def _mm_halves_body(a_ref, w_ref, oL_ref, oR_ref):
    h = jnp.dot(a_ref[...], w_ref[...], preferred_element_type=jnp.float32)
    half = oL_ref.shape[1]
    oL_ref[...] = h[:, :half]
    oR_ref[...] = h[:, half:]


def _mm_halves(a, w, bm):
    m, k = a.shape
    _, n = w.shape
    half = n // 2
    return pl.pallas_call(
        _mm_halves_body,
        grid=(m // bm,),
        in_specs=[
            pl.BlockSpec((bm, k), lambda i: (i, 0)),
            pl.BlockSpec((k, n), lambda i: (0, 0)),
        ],
        out_specs=[pl.BlockSpec((bm, half), lambda i: (i, 0))] * 2,
        out_shape=[jax.ShapeDtypeStruct((m, half), jnp.float32)] * 2,
    )(a, w)


def _epi_body(aL_ref, aR_ref, hL_ref, hR_ref, d2_ref, b_ref, o_ref):
    d2 = d2_ref[...]
    half = aL_ref.shape[1]
    o_ref[:, :half] = jnp.maximum(
        aL_ref[...] + d2 * hL_ref[...] + b_ref[:, :half], 0.0)
    o_ref[:, half:] = jnp.maximum(
        aR_ref[...] + d2 * hR_ref[...] + b_ref[:, half:], 0.0)


def _epilogue(aL, aR, hL, hR, d2, b, bm):
    m, half = aL.shape
    nn = 2 * half
    bspec = pl.BlockSpec((bm, half), lambda i: (i, 0))
    return pl.pallas_call(
        _epi_body,
        grid=(m // bm,),
        in_specs=[
            bspec, bspec, bspec, bspec,
            pl.BlockSpec((bm, 1), lambda i: (i, 0)),
            pl.BlockSpec((1, nn), lambda i: (0, 0)),
        ],
        out_specs=pl.BlockSpec((bm, nn), lambda i: (i, 0)),
        out_shape=jax.ShapeDtypeStruct((m, nn), jnp.float32),
    )(aL, aR, hL, hR, d2, b)


def _mm(a, b, bm, bk):
    m, k = a.shape
    _, n = b.shape
    return pl.pallas_call(
        _mm_body,
        grid=(m // bm, k // bk),
        in_specs=[
            pl.BlockSpec((bm, bk), lambda i, j: (i, j)),
            pl.BlockSpec((bk, n), lambda i, j: (j, 0)),
        ],
        out_specs=pl.BlockSpec((bm, n), lambda i, j: (i, 0)),
        out_shape=jax.ShapeDtypeStruct((m, n), jnp.float32),
    )(a, b)


def kernel(x, edge_index, edge_weights, W1, b1, W2, b2, Wlin, blin):
    src = edge_index[0]
    dst = edge_index[1]
    ew = edge_weights
    pad = EP - src.shape[0]
    srcp = jnp.pad(src, (0, pad))
    dstp = jnp.pad(dst, (0, pad))
    ewp = jnp.pad(ew, (0, pad))

    partials = _deg_partials(dstp, ewp).reshape(NTILES, NP)
    dis_row, d2_row = _dis_from_partials(partials)
    dis = dis_row[0]
    d2 = d2_row[0][:, None]

    norm = _edge_norm(srcp, dstp, ewp, dis)

    norm_bits = lax.bitcast_convert_type(norm, jnp.int32)
    edata = jnp.stack(
        [srcp.reshape(16, _NBLK, _BLK),
         dstp.reshape(16, _NBLK, _BLK),
         norm_bits.reshape(16, _NBLK, _BLK)], axis=2,
    ).reshape(16 * _NBLK, 3, _BLK)

    zeros128 = jnp.zeros((NP, 128), jnp.float32)
    zeros32 = jnp.zeros((NP, 32), jnp.float32)
    xp = jnp.pad(x, ((0, NP - N), (0, 0)))

    h1L, h1R = _mm_halves(xp, W1, bm=2048)
    a1L, a1R = _AGG128(h1L, h1R, edata, zeros128)
    z1 = _epilogue(a1L, a1R, h1L, h1R, d2, b1.reshape(1, -1), bm=2048)

    h2L, h2R = _mm_halves(z1, W2, bm=2048)
    a2L, a2R = _AGG32(h2L, h2R, edata.reshape(16 * _NBLK * 3, _BLK), zeros32)
    z2 = _epilogue(a2L, a2R, h2L, h2R, d2, b2.reshape(1, -1), bm=2048)

    out = _mm(z2[:N].reshape(1, -1), Wlin, bm=1, bk=12800) + blin
    return out.reshape(1, 64)


# speedup vs baseline: 2.8451x; 2.8451x over previous
import functools

import jax
import jax.numpy as jnp
from jax import lax
from jax.experimental import pallas as pl
from jax.experimental.pallas import tpu as pltpu
from jax.experimental.pallas import tpu_sc as plsc

N = 10000
NP = 10240
EP = 163840
NTILES = 32
EDGES_PER_TILE = EP // NTILES

_MESH = plsc.VectorSubcoreMesh(core_axis_name="c", subcore_axis_name="s")
_GDNUMS = lax.GatherDimensionNumbers(
    offset_dims=(), collapsed_slice_dims=(0,), start_index_map=(0,))


def _bcast_lane(vec, j):
    idx = jnp.full((16, 1), j, jnp.int32)
    return lax.gather(vec, idx, _GDNUMS, (1,),
                      mode=lax.GatherScatterMode.PROMISE_IN_BOUNDS)
_SC_PARAMS = pltpu.CompilerParams(needs_layout_passes=False,
                                  use_tc_tiling_on_sc=False)


@functools.partial(
    pl.kernel,
    mesh=_MESH,
    out_type=jax.ShapeDtypeStruct((NTILES * NP,), jnp.float32),
    compiler_params=_SC_PARAMS,
    scratch_types=[
        pltpu.VMEM((EDGES_PER_TILE,), jnp.int32),
        pltpu.VMEM((EDGES_PER_TILE,), jnp.float32),
        pltpu.VMEM((NP,), jnp.float32),
    ],
)
def _deg_partials(dst_hbm, ew_hbm, part_hbm, dstv, ewv, degv):
    wid = lax.axis_index("c") * 16 + lax.axis_index("s")
    base = wid * EDGES_PER_TILE
    pltpu.sync_copy(dst_hbm.at[pl.ds(base, EDGES_PER_TILE)], dstv)
    pltpu.sync_copy(ew_hbm.at[pl.ds(base, EDGES_PER_TILE)], ewv)

    def _zero(i, _):
        degv[pl.ds(i * 16, 16)] = jnp.zeros((16,), jnp.float32)
        return 0

    lax.fori_loop(0, NP // 16, _zero, 0)

    def _acc(k, _):
        idx = dstv[pl.ds(k * 16, 16)]
        w = ewv[pl.ds(k * 16, 16)]
        plsc.addupdate_scatter(degv, [idx], w)
        return 0

    lax.fori_loop(0, EDGES_PER_TILE // 16, _acc, 0)
    pltpu.sync_copy(degv, part_hbm.at[pl.ds(wid * NP, NP)])


def _dis_body(p_ref, dis_ref, d2_ref):
    deg = 1.0 + jnp.sum(p_ref[...], axis=0, keepdims=True)
    d2_ref[...] = 1.0 / deg
    dis_ref[...] = lax.rsqrt(deg)


def _dis_from_partials(partials_padded):
    return pl.pallas_call(
        _dis_body,
        out_shape=(
            jax.ShapeDtypeStruct((1, NP), jnp.float32),
            jax.ShapeDtypeStruct((1, NP), jnp.float32),
        ),
    )(partials_padded)


@functools.partial(
    pl.kernel,
    mesh=_MESH,
    out_type=jax.ShapeDtypeStruct((EP,), jnp.float32),
    compiler_params=_SC_PARAMS,
    scratch_types=[
        pltpu.VMEM((NP,), jnp.float32),
        pltpu.VMEM((EDGES_PER_TILE,), jnp.int32),
        pltpu.VMEM((EDGES_PER_TILE,), jnp.int32),
        pltpu.VMEM((EDGES_PER_TILE,), jnp.float32),
        pltpu.VMEM((EDGES_PER_TILE,), jnp.float32),
    ],
)
def _edge_norm(src_hbm, dst_hbm, ew_hbm, dis_hbm, norm_hbm,
               disv, srcv, dstv, ewv, normv):
    wid = lax.axis_index("c") * 16 + lax.axis_index("s")
    base = wid * EDGES_PER_TILE
    pltpu.sync_copy(dis_hbm, disv)
    pltpu.sync_copy(src_hbm.at[pl.ds(base, EDGES_PER_TILE)], srcv)
    pltpu.sync_copy(dst_hbm.at[pl.ds(base, EDGES_PER_TILE)], dstv)
    pltpu.sync_copy(ew_hbm.at[pl.ds(base, EDGES_PER_TILE)], ewv)

    def _body(k, _):
        s = srcv[pl.ds(k * 16, 16)]
        d = dstv[pl.ds(k * 16, 16)]
        w = ewv[pl.ds(k * 16, 16)]
        a = plsc.load_gather(disv, [s])
        b = plsc.load_gather(disv, [d])
        normv[pl.ds(k * 16, 16)] = a * w * b
        return 0

    lax.fori_loop(0, EDGES_PER_TILE // 16, _body, 0)
    pltpu.sync_copy(normv, norm_hbm.at[pl.ds(base, EDGES_PER_TILE)])


_NBLK = 80
_BLK = 128
_ROWS_PER_TILE = NP // 16


def _make_agg128():
    F = 128

    @functools.partial(
        pl.kernel,
        mesh=_MESH,
        out_type=(
            jax.ShapeDtypeStruct((NP, F), jnp.float32),
            jax.ShapeDtypeStruct((NP, F), jnp.float32),
        ),
        compiler_params=_SC_PARAMS,
        scratch_types=[
            pltpu.VMEM((4, 3, _BLK), jnp.int32),
            pltpu.VMEM((_BLK, F), jnp.float32),
            pltpu.VMEM((_BLK, F), jnp.float32),
            pltpu.VMEM_SHARED((NP, F), jnp.float32),
            pltpu.SemaphoreType.DMA,
            pltpu.SemaphoreType.DMA,
            pltpu.SemaphoreType.DMA,
            pltpu.SemaphoreType.DMA,
            pltpu.SemaphoreType.DMA,
            pltpu.SemaphoreType.DMA,
            pltpu.SemaphoreType.DMA,
            pltpu.SemaphoreType.DMA,
        ],
    )
    def _agg(hL, hR, edata, zeros_hbm, outL, outR,
             eb, rows0, rows1, acc, e0, e1, e2, e3, g0, g1, s0, s1):
        cid = lax.axis_index("c")
        sid = lax.axis_index("s")
        rsl = pl.ds(sid * _ROWS_PER_TILE, _ROWS_PER_TILE)
        pltpu.sync_copy(zeros_hbm.at[rsl], acc.at[rsl])
        plsc.subcore_barrier()

        rows = (rows0, rows1)
        esem = (e0, e1, e2, e3)
        gsem = (g0, g1)
        ssem = (s0, s1)
        base = sid * _NBLK

        def _process(h_hbm):
            def estart(j, s):
                pltpu.async_copy(edata.at[base + j], eb.at[s], esem[s])

            def ewait(s):
                pltpu.make_async_copy(
                    edata.at[base], eb.at[s], esem[s]).wait()

            def gstart(r, s):
                pltpu.async_copy(h_hbm.at[eb.at[s, 0]], rows[r], gsem[r])

            def gwait(r, s):
                pltpu.make_async_copy(
                    h_hbm.at[eb.at[s, 0]], rows[r], gsem[r]).wait()

            def sstart(r, s):
                pltpu.async_copy(rows[r], acc.at[eb.at[s, 1]], ssem[r],
                                 add=True)

            def swait(r, s):
                pltpu.make_async_copy(
                    rows[r], acc.at[eb.at[s, 1]], ssem[r]).wait()

            def scale(r, s):
                rp = rows[r]

                def _grp(g, _):
                    w16 = plsc.bitcast(eb[s, 2, pl.ds(g * 16, 16)],
                                       jnp.float32)
                    e0 = g * 16
                    for j in range(16):
                        w = _bcast_lane(w16, j)
                        for v in range(F // 16):
                            sl = pl.ds(v * 16, 16)
                            rp[e0 + j, sl] = rp[e0 + j, sl] * w
                    return 0

                lax.fori_loop(0, _BLK // 16, _grp, 0)

            for s in range(3):
                estart(s, s)
            ewait(0)
            gstart(0, 0)

            def _outer(i, _):
                for jp in range(4):
                    r = jp % 2
                    q = 1 - r
                    j = 4 * i + jp
                    if jp == 0:
                        @pl.when(i >= 1)
                        def _wq():
                            swait(q, (jp + 3) % 4)
                    else:
                        swait(q, (jp + 3) % 4)
                    if jp == 0:
                        estart(j + 3, 3)
                    else:
                        @pl.when(j + 3 < _NBLK)
                        def _st():
                            estart(j + 3, (jp + 3) % 4)
                    if jp < 3:
                        ewait(jp + 1)
                        gstart(q, jp + 1)
                    else:
                        @pl.when(i < _NBLK // 4 - 1)
                        def _g0():
                            ewait(0)
                            gstart(q, 0)
                    gwait(r, jp)
                    scale(r, jp)
                    sstart(r, jp)
                return 0

            lax.fori_loop(0, _NBLK // 4, _outer, 0)
            swait((_NBLK - 1) % 2, (_NBLK - 1) % 4)

        @pl.when(cid == 0)
        def _left():
            _process(hL)

        @pl.when(cid == 1)
        def _right():
            _process(hR)

        plsc.subcore_barrier()

        @pl.when(cid == 0)
        def _outl():
            pltpu.sync_copy(acc.at[rsl], outL.at[rsl])

        @pl.when(cid == 1)
        def _outr():
            pltpu.sync_copy(acc.at[rsl], outR.at[rsl])

    return _agg


def _make_agg32():
    F = 32
    ROWS3 = _NBLK * 3

    @functools.partial(
        pl.kernel,
        mesh=_MESH,
        out_type=(
            jax.ShapeDtypeStruct((NP, F), jnp.float32),
            jax.ShapeDtypeStruct((NP, F), jnp.float32),
        ),
        compiler_params=_SC_PARAMS,
        scratch_types=[
            pltpu.VMEM((ROWS3, _BLK), jnp.int32),
            pltpu.VMEM((_BLK, F), jnp.float32),
            pltpu.VMEM((_BLK, F), jnp.float32),
            pltpu.VMEM_SHARED((NP, F), jnp.float32),
            pltpu.SemaphoreType.DMA,
            pltpu.SemaphoreType.DMA,
            pltpu.SemaphoreType.DMA,
            pltpu.SemaphoreType.DMA,
        ],
    )
    def _agg(hL, hR, edata2, zeros_hbm, outL, outR,
             eball, rows0, rows1, acc, g0, g1, s0, s1):
        cid = lax.axis_index("c")
        sid = lax.axis_index("s")
        rsl = pl.ds(sid * _ROWS_PER_TILE, _ROWS_PER_TILE)
        pltpu.sync_copy(zeros_hbm.at[rsl], acc.at[rsl])
        pltpu.sync_copy(edata2.at[pl.ds(sid * ROWS3, ROWS3)], eball)
        plsc.subcore_barrier()

        rows = (rows0, rows1)
        gsem = (g0, g1)
        ssem = (s0, s1)

        def _process(h_hbm):
            def gstart(j, r):
                pltpu.async_copy(h_hbm.at[eball.at[3 * j]], rows[r], gsem[r])

            def gwait(r):
                pltpu.make_async_copy(
                    h_hbm.at[eball.at[0]], rows[r], gsem[r]).wait()

            def sstart(j, r):
                pltpu.async_copy(rows[r], acc.at[eball.at[3 * j + 1]],
                                 ssem[r], add=True)

            def swait(r):
                pltpu.make_async_copy(
                    rows[r], acc.at[eball.at[1]], ssem[r]).wait()

            def scale(j, r):
                rp = rows[r]

                def _grp(g, _):
                    w16 = plsc.bitcast(eball[3 * j + 2, pl.ds(g * 16, 16)],
                                       jnp.float32)
                    e0 = g * 16
                    for jj in range(16):
                        w = _bcast_lane(w16, jj)
                        for v in range(F // 16):
                            sl = pl.ds(v * 16, 16)
                            rp[e0 + jj, sl] = rp[e0 + jj, sl] * w
                    return 0

                lax.fori_loop(0, _BLK // 16, _grp, 0)

            gstart(0, 0)

            def _outer(i, _):
                @pl.when(i >= 1)
                def _w1():
                    swait(1)
                gstart(2 * i + 1, 1)
                gwait(0)
                scale(2 * i, 0)
                sstart(2 * i, 0)
                swait(0)
                @pl.when(i < _NBLK // 2 - 1)
                def _g0():
                    gstart(2 * i + 2, 0)
                gwait(1)
                scale(2 * i + 1, 1)
                sstart(2 * i + 1, 1)
                return 0

            lax.fori_loop(0, _NBLK // 2, _outer, 0)
            swait(1)

        @pl.when(cid == 0)
        def _left():
            _process(hL)

        @pl.when(cid == 1)
        def _right():
            _process(hR)

        plsc.subcore_barrier()

        @pl.when(cid == 0)
        def _outl():
            pltpu.sync_copy(acc.at[rsl], outL.at[rsl])

        @pl.when(cid == 1)
        def _outr():
            pltpu.sync_copy(acc.at[rsl], outR.at[rsl])

    return _agg


_AGG128 = _make_agg128()
_AGG32 = _make_agg32()


def _mm_body(a_ref, b_ref, o_ref):
    @pl.when(pl.program_id(1) == 0)
    def _init():
        o_ref[...] = jnp.zeros_like(o_ref)

    o_ref[...] += jnp.dot(a_ref[...], b_ref[...],
                          preferred_element_type=jnp.float32)


def _mm_halves_body(a_ref, w_ref, oL_ref, oR_ref):
    h = jnp.dot(a_ref[...], w_ref[...], preferred_element_type=jnp.float32)
    half = oL_ref.shape[1]
    oL_ref[...] = h[:, :half]
    oR_ref[...] = h[:, half:]


def _mm_halves(a, w, bm):
    m, k = a.shape
    _, n = w.shape
    half = n // 2
    return pl.pallas_call(
        _mm_halves_body,
        grid=(m // bm,),
        in_specs=[
            pl.BlockSpec((bm, k), lambda i: (i, 0)),
            pl.BlockSpec((k, n), lambda i: (0, 0)),
        ],
        out_specs=[pl.BlockSpec((bm, half), lambda i: (i, 0))] * 2,
        out_shape=[jax.ShapeDtypeStruct((m, half), jnp.float32)] * 2,
    )(a, w)


def _epi_body(aL_ref, aR_ref, hL_ref, hR_ref, d2_ref, b_ref, o_ref):
    d2 = d2_ref[...]
    half = aL_ref.shape[1]
    o_ref[:, :half] = jnp.maximum(
        aL_ref[...] + d2 * hL_ref[...] + b_ref[:, :half], 0.0)
    o_ref[:, half:] = jnp.maximum(
        aR_ref[...] + d2 * hR_ref[...] + b_ref[:, half:], 0.0)


def _epilogue(aL, aR, hL, hR, d2, b, bm):
    m, half = aL.shape
    nn = 2 * half
    bspec = pl.BlockSpec((bm, half), lambda i: (i, 0))
    return pl.pallas_call(
        _epi_body,
        grid=(m // bm,),
        in_specs=[
            bspec, bspec, bspec, bspec,
            pl.BlockSpec((bm, 1), lambda i: (i, 0)),
            pl.BlockSpec((1, nn), lambda i: (0, 0)),
        ],
        out_specs=pl.BlockSpec((bm, nn), lambda i: (i, 0)),
        out_shape=jax.ShapeDtypeStruct((m, nn), jnp.float32),
    )(aL, aR, hL, hR, d2, b)


def _mm(a, b, bm, bk):
    m, k = a.shape
    _, n = b.shape
    return pl.pallas_call(
        _mm_body,
        grid=(m // bm, k // bk),
        in_specs=[
            pl.BlockSpec((bm, bk), lambda i, j: (i, j)),
            pl.BlockSpec((bk, n), lambda i, j: (j, 0)),
        ],
        out_specs=pl.BlockSpec((bm, n), lambda i, j: (i, 0)),
        out_shape=jax.ShapeDtypeStruct((m, n), jnp.float32),
    )(a, b)


def kernel(x, edge_index, edge_weights, W1, b1, W2, b2, Wlin, blin):
    src = edge_index[0]
    dst = edge_index[1]
    ew = edge_weights
    pad = EP - src.shape[0]
    srcp = jnp.pad(src, (0, pad))
    dstp = jnp.pad(dst, (0, pad))
    ewp = jnp.pad(ew, (0, pad))

    partials = _deg_partials(dstp, ewp).reshape(NTILES, NP)
    dis_row, d2_row = _dis_from_partials(partials)
    dis = dis_row[0]
    d2 = d2_row[0][:, None]

    norm = _edge_norm(srcp, dstp, ewp, dis)

    norm_bits = lax.bitcast_convert_type(norm, jnp.int32)
    edata = jnp.stack(
        [srcp.reshape(16, _NBLK, _BLK),
         dstp.reshape(16, _NBLK, _BLK),
         norm_bits.reshape(16, _NBLK, _BLK)], axis=2,
    ).reshape(16 * _NBLK, 3, _BLK)

    zeros128 = jnp.zeros((NP, 128), jnp.float32)
    zeros32 = jnp.zeros((NP, 32), jnp.float32)
    xp = jnp.pad(x, ((0, NP - N), (0, 0)))

    h1L, h1R = _mm_halves(xp, W1, bm=2048)
    a1L, a1R = _AGG128(h1L, h1R, edata, zeros128)
    z1 = _epilogue(a1L, a1R, h1L, h1R, d2, b1.reshape(1, -1), bm=2048)

    h2L, h2R = _mm_halves(z1, W2, bm=2048)
    a2L, a2R = _AGG32(h2L, h2R, edata.reshape(16 * _NBLK * 3, _BLK), zeros32)
    z2 = _epilogue(a2L, a2R, h2L, h2R, d2, b2.reshape(1, -1), bm=2048)

    out = _mm(z2[:N].reshape(1, -1), Wlin, bm=1, bk=12800) + blin
    return out.reshape(1, 64)


# speedup vs baseline: 2.8847x; 1.0139x over previous
import functools

import jax
import jax.numpy as jnp
from jax import lax
from jax.experimental import pallas as pl
from jax.experimental.pallas import tpu as pltpu
from jax.experimental.pallas import tpu_sc as plsc

N = 10000
NP = 10240
EP = 163840
NTILES = 32
EDGES_PER_TILE = EP // NTILES

_MESH = plsc.VectorSubcoreMesh(core_axis_name="c", subcore_axis_name="s")
_GDNUMS = lax.GatherDimensionNumbers(
    offset_dims=(), collapsed_slice_dims=(0,), start_index_map=(0,))


def _bcast_lane(vec, j):
    idx = jnp.full((16, 1), j, jnp.int32)
    return lax.gather(vec, idx, _GDNUMS, (1,),
                      mode=lax.GatherScatterMode.PROMISE_IN_BOUNDS)
_SC_PARAMS = pltpu.CompilerParams(needs_layout_passes=False,
                                  use_tc_tiling_on_sc=False)


@functools.partial(
    pl.kernel,
    mesh=_MESH,
    out_type=jax.ShapeDtypeStruct((NTILES * NP,), jnp.float32),
    compiler_params=_SC_PARAMS,
    scratch_types=[
        pltpu.VMEM((EDGES_PER_TILE,), jnp.int32),
        pltpu.VMEM((EDGES_PER_TILE,), jnp.float32),
        pltpu.VMEM((NP,), jnp.float32),
    ],
)
def _deg_partials(dst_hbm, ew_hbm, part_hbm, dstv, ewv, degv):
    wid = lax.axis_index("c") * 16 + lax.axis_index("s")
    base = wid * EDGES_PER_TILE
    pltpu.sync_copy(dst_hbm.at[pl.ds(base, EDGES_PER_TILE)], dstv)
    pltpu.sync_copy(ew_hbm.at[pl.ds(base, EDGES_PER_TILE)], ewv)

    def _zero(i, _):
        degv[pl.ds(i * 16, 16)] = jnp.zeros((16,), jnp.float32)
        return 0

    lax.fori_loop(0, NP // 16, _zero, 0)

    def _acc(k, _):
        idx = dstv[pl.ds(k * 16, 16)]
        w = ewv[pl.ds(k * 16, 16)]
        plsc.addupdate_scatter(degv, [idx], w)
        return 0

    lax.fori_loop(0, EDGES_PER_TILE // 16, _acc, 0)
    pltpu.sync_copy(degv, part_hbm.at[pl.ds(wid * NP, NP)])


def _dis_body(p_ref, dis_ref, d2_ref):
    deg = 1.0 + jnp.sum(p_ref[...], axis=0, keepdims=True)
    d2_ref[...] = 1.0 / deg
    dis_ref[...] = lax.rsqrt(deg)


def _dis_from_partials(partials_padded):
    return pl.pallas_call(
        _dis_body,
        out_shape=(
            jax.ShapeDtypeStruct((1, NP), jnp.float32),
            jax.ShapeDtypeStruct((1, NP), jnp.float32),
        ),
    )(partials_padded)


@functools.partial(
    pl.kernel,
    mesh=_MESH,
    out_type=jax.ShapeDtypeStruct((EP,), jnp.float32),
    compiler_params=_SC_PARAMS,
    scratch_types=[
        pltpu.VMEM((NP,), jnp.float32),
        pltpu.VMEM((EDGES_PER_TILE,), jnp.int32),
        pltpu.VMEM((EDGES_PER_TILE,), jnp.int32),
        pltpu.VMEM((EDGES_PER_TILE,), jnp.float32),
        pltpu.VMEM((EDGES_PER_TILE,), jnp.float32),
    ],
)
def _edge_norm(src_hbm, dst_hbm, ew_hbm, dis_hbm, norm_hbm,
               disv, srcv, dstv, ewv, normv):
    wid = lax.axis_index("c") * 16 + lax.axis_index("s")
    base = wid * EDGES_PER_TILE
    pltpu.sync_copy(dis_hbm, disv)
    pltpu.sync_copy(src_hbm.at[pl.ds(base, EDGES_PER_TILE)], srcv)
    pltpu.sync_copy(dst_hbm.at[pl.ds(base, EDGES_PER_TILE)], dstv)
    pltpu.sync_copy(ew_hbm.at[pl.ds(base, EDGES_PER_TILE)], ewv)

    def _body(k, _):
        s = srcv[pl.ds(k * 16, 16)]
        d = dstv[pl.ds(k * 16, 16)]
        w = ewv[pl.ds(k * 16, 16)]
        a = plsc.load_gather(disv, [s])
        b = plsc.load_gather(disv, [d])
        normv[pl.ds(k * 16, 16)] = a * w * b
        return 0

    lax.fori_loop(0, EDGES_PER_TILE // 16, _body, 0)
    pltpu.sync_copy(normv, norm_hbm.at[pl.ds(base, EDGES_PER_TILE)])


_NB1, _B1 = 64, 160
_NB2, _B2 = 40, 256
_ROWS_PER_TILE = NP // 16


def _make_agg128():
    F = 128

    @functools.partial(
        pl.kernel,
        mesh=_MESH,
        out_type=(
            jax.ShapeDtypeStruct((NP, F), jnp.float32),
            jax.ShapeDtypeStruct((NP, F), jnp.float32),
        ),
        compiler_params=_SC_PARAMS,
        scratch_types=[
            pltpu.VMEM((4, 3, _B1), jnp.int32),
            pltpu.VMEM((_B1, F), jnp.float32),
            pltpu.VMEM((_B1, F), jnp.float32),
            pltpu.VMEM_SHARED((NP, F), jnp.float32),
            pltpu.SemaphoreType.DMA,
            pltpu.SemaphoreType.DMA,
            pltpu.SemaphoreType.DMA,
            pltpu.SemaphoreType.DMA,
            pltpu.SemaphoreType.DMA,
            pltpu.SemaphoreType.DMA,
            pltpu.SemaphoreType.DMA,
            pltpu.SemaphoreType.DMA,
        ],
    )
    def _agg(hL, hR, edata, zeros_hbm, outL, outR,
             eb, rows0, rows1, acc, e0, e1, e2, e3, g0, g1, s0, s1):
        cid = lax.axis_index("c")
        sid = lax.axis_index("s")
        rsl = pl.ds(sid * _ROWS_PER_TILE, _ROWS_PER_TILE)
        pltpu.sync_copy(zeros_hbm.at[rsl], acc.at[rsl])
        plsc.subcore_barrier()

        rows = (rows0, rows1)
        esem = (e0, e1, e2, e3)
        gsem = (g0, g1)
        ssem = (s0, s1)
        base = sid * _NB1

        def _process(h_hbm):
            def estart(j, s):
                pltpu.async_copy(edata.at[base + j], eb.at[s], esem[s])

            def ewait(s):
                pltpu.make_async_copy(
                    edata.at[base], eb.at[s], esem[s]).wait()

            def gstart(r, s):
                pltpu.async_copy(h_hbm.at[eb.at[s, 0]], rows[r], gsem[r])

            def gwait(r, s):
                pltpu.make_async_copy(
                    h_hbm.at[eb.at[s, 0]], rows[r], gsem[r]).wait()

            def sstart(r, s):
                pltpu.async_copy(rows[r], acc.at[eb.at[s, 1]], ssem[r],
                                 add=True)

            def swait(r, s):
                pltpu.make_async_copy(
                    rows[r], acc.at[eb.at[s, 1]], ssem[r]).wait()

            def scale(r, s):
                rp = rows[r]

                def _grp(g, _):
                    w16 = plsc.bitcast(eb[s, 2, pl.ds(g * 16, 16)],
                                       jnp.float32)
                    e0 = g * 16
                    for j in range(16):
                        w = _bcast_lane(w16, j)
                        for v in range(F // 16):
                            sl = pl.ds(v * 16, 16)
                            rp[e0 + j, sl] = rp[e0 + j, sl] * w
                    return 0

                lax.fori_loop(0, _B1 // 16, _grp, 0)

            for s in range(3):
                estart(s, s)
            ewait(0)
            gstart(0, 0)

            def _outer(i, _):
                for jp in range(4):
                    r = jp % 2
                    q = 1 - r
                    j = 4 * i + jp
                    if jp == 0:
                        @pl.when(i >= 1)
                        def _wq():
                            swait(q, (jp + 3) % 4)
                    else:
                        swait(q, (jp + 3) % 4)
                    if jp == 0:
                        estart(j + 3, 3)
                    else:
                        @pl.when(j + 3 < _NB1)
                        def _st():
                            estart(j + 3, (jp + 3) % 4)
                    if jp < 3:
                        ewait(jp + 1)
                        gstart(q, jp + 1)
                    else:
                        @pl.when(i < _NB1 // 4 - 1)
                        def _g0():
                            ewait(0)
                            gstart(q, 0)
                    gwait(r, jp)
                    scale(r, jp)
                    sstart(r, jp)
                return 0

            lax.fori_loop(0, _NB1 // 4, _outer, 0)
            swait((_NB1 - 1) % 2, (_NB1 - 1) % 4)

        @pl.when(cid == 0)
        def _left():
            _process(hL)

        @pl.when(cid == 1)
        def _right():
            _process(hR)

        plsc.subcore_barrier()

        @pl.when(cid == 0)
        def _outl():
            pltpu.sync_copy(acc.at[rsl], outL.at[rsl])

        @pl.when(cid == 1)
        def _outr():
            pltpu.sync_copy(acc.at[rsl], outR.at[rsl])

    return _agg


def _make_agg32():
    F = 32
    ROWS3 = _NB2 * 3

    @functools.partial(
        pl.kernel,
        mesh=_MESH,
        out_type=(
            jax.ShapeDtypeStruct((NP, F), jnp.float32),
            jax.ShapeDtypeStruct((NP, F), jnp.float32),
        ),
        compiler_params=_SC_PARAMS,
        scratch_types=[
            pltpu.VMEM((ROWS3, _B2), jnp.int32),
            pltpu.VMEM((_B2, F), jnp.float32),
            pltpu.VMEM((_B2, F), jnp.float32),
            pltpu.VMEM_SHARED((NP, F), jnp.float32),
            pltpu.SemaphoreType.DMA,
            pltpu.SemaphoreType.DMA,
            pltpu.SemaphoreType.DMA,
            pltpu.SemaphoreType.DMA,
        ],
    )
    def _agg(hL, hR, edata2, zeros_hbm, outL, outR,
             eball, rows0, rows1, acc, g0, g1, s0, s1):
        cid = lax.axis_index("c")
        sid = lax.axis_index("s")
        rsl = pl.ds(sid * _ROWS_PER_TILE, _ROWS_PER_TILE)
        pltpu.sync_copy(zeros_hbm.at[rsl], acc.at[rsl])
        pltpu.sync_copy(edata2.at[pl.ds(sid * ROWS3, ROWS3)], eball)
        plsc.subcore_barrier()

        rows = (rows0, rows1)
        gsem = (g0, g1)
        ssem = (s0, s1)

        def _process(h_hbm):
            def gstart(j, r):
                pltpu.async_copy(h_hbm.at[eball.at[3 * j]], rows[r], gsem[r])

            def gwait(r):
                pltpu.make_async_copy(
                    h_hbm.at[eball.at[0]], rows[r], gsem[r]).wait()

            def sstart(j, r):
                pltpu.async_copy(rows[r], acc.at[eball.at[3 * j + 1]],
                                 ssem[r], add=True)

            def swait(r):
                pltpu.make_async_copy(
                    rows[r], acc.at[eball.at[1]], ssem[r]).wait()

            def scale(j, r):
                rp = rows[r]

                def _grp(g, _):
                    w16 = plsc.bitcast(eball[3 * j + 2, pl.ds(g * 16, 16)],
                                       jnp.float32)
                    e0 = g * 16
                    for jj in range(16):
                        w = _bcast_lane(w16, jj)
                        for v in range(F // 16):
                            sl = pl.ds(v * 16, 16)
                            rp[e0 + jj, sl] = rp[e0 + jj, sl] * w
                    return 0

                lax.fori_loop(0, _B2 // 16, _grp, 0)

            gstart(0, 0)

            def _outer(i, _):
                @pl.when(i >= 1)
                def _w1():
                    swait(1)
                gstart(2 * i + 1, 1)
                gwait(0)
                scale(2 * i, 0)
                sstart(2 * i, 0)
                swait(0)
                @pl.when(i < _NB2 // 2 - 1)
                def _g0():
                    gstart(2 * i + 2, 0)
                gwait(1)
                scale(2 * i + 1, 1)
                sstart(2 * i + 1, 1)
                return 0

            lax.fori_loop(0, _NB2 // 2, _outer, 0)
            swait(1)

        @pl.when(cid == 0)
        def _left():
            _process(hL)

        @pl.when(cid == 1)
        def _right():
            _process(hR)

        plsc.subcore_barrier()

        @pl.when(cid == 0)
        def _outl():
            pltpu.sync_copy(acc.at[rsl], outL.at[rsl])

        @pl.when(cid == 1)
        def _outr():
            pltpu.sync_copy(acc.at[rsl], outR.at[rsl])

    return _agg


_AGG128 = _make_agg128()
_AGG32 = _make_agg32()


def _mm_body(a_ref, b_ref, o_ref):
    @pl.when(pl.program_id(1) == 0)
    def _init():
        o_ref[...] = jnp.zeros_like(o_ref)

    o_ref[...] += jnp.dot(a_ref[...], b_ref[...],
                          preferred_element_type=jnp.float32)


def _mm_halves_body(a_ref, w_ref, oL_ref, oR_ref):
    h = jnp.dot(a_ref[...], w_ref[...], preferred_element_type=jnp.float32)
    half = oL_ref.shape[1]
    oL_ref[...] = h[:, :half]
    oR_ref[...] = h[:, half:]


def _mm_halves(a, w, bm):
    m, k = a.shape
    _, n = w.shape
    half = n // 2
    return pl.pallas_call(
        _mm_halves_body,
        grid=(m // bm,),
        in_specs=[
            pl.BlockSpec((bm, k), lambda i: (i, 0)),
            pl.BlockSpec((k, n), lambda i: (0, 0)),
        ],
        out_specs=[pl.BlockSpec((bm, half), lambda i: (i, 0))] * 2,
        out_shape=[jax.ShapeDtypeStruct((m, half), jnp.float32)] * 2,
    )(a, w)


def _epi_body(aL_ref, aR_ref, hL_ref, hR_ref, d2_ref, b_ref, o_ref):
    d2 = d2_ref[...]
    half = aL_ref.shape[1]
    o_ref[:, :half] = jnp.maximum(
        aL_ref[...] + d2 * hL_ref[...] + b_ref[:, :half], 0.0)
    o_ref[:, half:] = jnp.maximum(
        aR_ref[...] + d2 * hR_ref[...] + b_ref[:, half:], 0.0)


def _epilogue(aL, aR, hL, hR, d2, b, bm):
    m, half = aL.shape
    nn = 2 * half
    bspec = pl.BlockSpec((bm, half), lambda i: (i, 0))
    return pl.pallas_call(
        _epi_body,
        grid=(m // bm,),
        in_specs=[
            bspec, bspec, bspec, bspec,
            pl.BlockSpec((bm, 1), lambda i: (i, 0)),
            pl.BlockSpec((1, nn), lambda i: (0, 0)),
        ],
        out_specs=pl.BlockSpec((bm, nn), lambda i: (i, 0)),
        out_shape=jax.ShapeDtypeStruct((m, nn), jnp.float32),
    )(aL, aR, hL, hR, d2, b)


def _mm(a, b, bm, bk):
    m, k = a.shape
    _, n = b.shape
    return pl.pallas_call(
        _mm_body,
        grid=(m // bm, k // bk),
        in_specs=[
            pl.BlockSpec((bm, bk), lambda i, j: (i, j)),
            pl.BlockSpec((bk, n), lambda i, j: (j, 0)),
        ],
        out_specs=pl.BlockSpec((bm, n), lambda i, j: (i, 0)),
        out_shape=jax.ShapeDtypeStruct((m, n), jnp.float32),
    )(a, b)


def kernel(x, edge_index, edge_weights, W1, b1, W2, b2, Wlin, blin):
    src = edge_index[0]
    dst = edge_index[1]
    ew = edge_weights
    pad = EP - src.shape[0]
    srcp = jnp.pad(src, (0, pad))
    dstp = jnp.pad(dst, (0, pad))
    ewp = jnp.pad(ew, (0, pad))

    partials = _deg_partials(dstp, ewp).reshape(NTILES, NP)
    dis_row, d2_row = _dis_from_partials(partials)
    dis = dis_row[0]
    d2 = d2_row[0][:, None]

    norm = _edge_norm(srcp, dstp, ewp, dis)

    norm_bits = lax.bitcast_convert_type(norm, jnp.int32)
    edata1 = jnp.stack(
        [srcp.reshape(16, _NB1, _B1),
         dstp.reshape(16, _NB1, _B1),
         norm_bits.reshape(16, _NB1, _B1)], axis=2,
    ).reshape(16 * _NB1, 3, _B1)
    edata2 = jnp.stack(
        [srcp.reshape(16, _NB2, _B2),
         dstp.reshape(16, _NB2, _B2),
         norm_bits.reshape(16, _NB2, _B2)], axis=2,
    ).reshape(16 * _NB2 * 3, _B2)

    zeros128 = jnp.zeros((NP, 128), jnp.float32)
    zeros32 = jnp.zeros((NP, 32), jnp.float32)
    xp = jnp.pad(x, ((0, NP - N), (0, 0)))

    h1L, h1R = _mm_halves(xp, W1, bm=2048)
    a1L, a1R = _AGG128(h1L, h1R, edata1, zeros128)
    z1 = _epilogue(a1L, a1R, h1L, h1R, d2, b1.reshape(1, -1), bm=2048)

    h2L, h2R = _mm_halves(z1, W2, bm=2048)
    a2L, a2R = _AGG32(h2L, h2R, edata2, zeros32)
    z2 = _epilogue(a2L, a2R, h2L, h2R, d2, b2.reshape(1, -1), bm=2048)

    out = _mm(z2[:N].reshape(1, -1), Wlin, bm=1, bk=12800) + blin
    return out.reshape(1, 64)
